# baseline (speedup 1.0000x reference)
import functools

import jax
import jax.numpy as jnp
from jax import lax
from jax.experimental import pallas as pl
from jax.experimental.pallas import tpu as pltpu
from jax.experimental.pallas import tpu_sc as plsc

NC, NS, LANES = 2, 16, 16
NW = NC * NS
B, L, D = 16384, 50, 128
MED = 1000
CH = 128
BPW = B // NW
NCHUNK = BPW // CH
NPC = 2 * NCHUNK


def _prep_codes(cur, prev):
    c = jnp.stack([cur, prev]).astype(jnp.int32)
    nb = c.shape[1]
    nchunk = nb // (NW * CH)
    c = c.reshape(2, NW, nchunk, CH, L)
    c = c.transpose(1, 0, 2, 4, 3)
    return c.reshape(NW, 2 * nchunk, L, CH)


@functools.cache
def _build_pool_sc(nb):
    bpw = nb // NW
    nchunk = bpw // CH
    npc = 2 * nchunk
    mesh = plsc.VectorSubcoreMesh(core_axis_name="c", subcore_axis_name="s",
                                  num_cores=NC, num_subcores=NS)

    @functools.partial(
        pl.kernel,
        out_type=jax.ShapeDtypeStruct((4, nb, D), jnp.float32),
        mesh=mesh,
        scratch_types=[
            pltpu.VMEM((2, L, CH), jnp.int32),
            pltpu.VMEM((2, CH, D), jnp.float32),
            pltpu.SemaphoreType.DMA,
            pltpu.SemaphoreType.DMA,
        ],
    )
    def pool_sc(diag_codes_h, proc_codes_h, diag_tab_h, proc_tab_h, out_h,
                idx_v, acc_v, sem0, sem1):
        wid = lax.axis_index("s") * NC + lax.axis_index("c")
        zeros = jnp.zeros((LANES,), jnp.float32)
        sems = (sem0, sem1)

        blocks = [(codes_h, tab_h, t, pc)
                  for t, (codes_h, tab_h) in enumerate(
                      ((diag_codes_h, diag_tab_h), (proc_codes_h, proc_tab_h)))
                  for pc in range(npc)]

        def fire_block(codes_h, tab_h, pc, buf):
            idx = idx_v.at[buf]
            acc = acc_v.at[buf]
            pltpu.sync_copy(codes_h.at[wid, pc], idx)

            def zrow(r, c2):
                for k in range(D // LANES):
                    acc_v[buf, r, pl.ds(k * LANES, LANES)] = zeros
                return c2

            lax.fori_loop(0, CH, zrow, 0)

            def fire(g, c2):
                pltpu.async_copy(tab_h.at[idx.at[g]], acc, sems[buf],
                                 add=True)
                return c2

            lax.fori_loop(0, L, fire, 0)

        def drain_block(tab_h, t, pc, buf):
            idx = idx_v.at[buf]
            acc = acc_v.at[buf]

            def drain(g, c2):
                pltpu.make_async_copy(tab_h.at[idx.at[0]], acc,
                                      sems[buf]).wait()
                return c2

            lax.fori_loop(0, L, drain, 0)
            p = pc // nchunk
            c = pc % nchunk
            base = wid * bpw + c * CH
            pltpu.sync_copy(acc, out_h.at[t * 2 + p, pl.ds(base, CH)])

        for i, (codes_h, tab_h, t, pc) in enumerate(blocks):
            fire_block(codes_h, tab_h, pc, i % 2)
            if i > 0:
                pcodes_h, ptab_h, pt, ppc = blocks[i - 1]
                drain_block(ptab_h, pt, ppc, (i - 1) % 2)
        lcodes_h, ltab_h, lt, lpc = blocks[-1]
        drain_block(ltab_h, lt, lpc, (len(blocks) - 1) % 2)

    return pool_sc


TILE = 2048


def _dot(a, b):
    return jnp.dot(a.astype(jnp.bfloat16), b.astype(jnp.bfloat16),
                   preferred_element_type=jnp.float32)


def _mlp_body(pooled_ref, w1t_ref, b1_ref, w2t_ref, b2_ref, w3t_ref, b3_ref,
              out_ref):
    ed = pooled_ref[0]
    pd = pooled_ref[1]
    ep = pooled_ref[2]
    pp = pooled_ref[3]
    cur = _dot(jnp.concatenate([ed, ep], axis=1), w1t_ref[...]) + b1_ref[...]
    prv = _dot(jnp.concatenate([pd, pp], axis=1), w1t_ref[...]) + b1_ref[...]
    rep = jnp.concatenate([cur, prv], axis=1)
    h = jnp.maximum(_dot(rep, w2t_ref[...]) + b2_ref[...], 0.0)
    out_ref[...] = jax.nn.sigmoid(_dot(h, w3t_ref[...]) + b3_ref[...])


def _mlp_tc(pooled, w1t, b1r, w2t, b2r, w3t, b3r):
    nb = pooled.shape[1]
    return pl.pallas_call(
        _mlp_body,
        grid=(nb // TILE,),
        in_specs=[
            pl.BlockSpec((4, TILE, D), lambda i: (0, i, 0)),
            pl.BlockSpec((2 * D, D), lambda i: (0, 0)),
            pl.BlockSpec((1, D), lambda i: (0, 0)),
            pl.BlockSpec((2 * D, 2 * D), lambda i: (0, 0)),
            pl.BlockSpec((1, 2 * D), lambda i: (0, 0)),
            pl.BlockSpec((2 * D, MED), lambda i: (0, 0)),
            pl.BlockSpec((1, MED), lambda i: (0, 0)),
        ],
        out_specs=pl.BlockSpec((TILE, MED), lambda i: (i, 0)),
        out_shape=jax.ShapeDtypeStruct((nb, MED), jnp.float32),
    )(pooled, w1t, b1r, w2t, b2r, w3t, b3r)


NSPLIT = 1


def kernel(diag_codes, proc_codes, prev_diag_codes, prev_proc_codes,
           diag_table, proc_table, W1, b1, W2, b2, W3, b3):
    h = B // NSPLIT
    w1t, b1r = W1.T, b1.reshape(1, -1)
    w2t, b2r = W2.T, b2.reshape(1, -1)
    w3t, b3r = W3.T, b3.reshape(1, -1)
    pooled_halves = []
    for s in range(NSPLIT):
        sl = slice(s * h, (s + 1) * h)
        dcodes = _prep_codes(diag_codes[sl], prev_diag_codes[sl])
        pcodes = _prep_codes(proc_codes[sl], prev_proc_codes[sl])
        pooled_halves.append(
            _build_pool_sc(h)(dcodes, pcodes, diag_table, proc_table))
    outs = [_mlp_tc(p, w1t, b1r, w2t, b2r, w3t, b3r) for p in pooled_halves]
    return outs[0] if NSPLIT == 1 else jnp.concatenate(outs, axis=0)


# speedup vs baseline: 1.0228x; 1.0228x over previous
import functools

import jax
import jax.numpy as jnp
from jax import lax
from jax.experimental import pallas as pl
from jax.experimental.pallas import tpu as pltpu
from jax.experimental.pallas import tpu_sc as plsc

NC, NS, LANES = 2, 16, 16
NW = NC * NS
B, L, D = 16384, 50, 128
MED = 1000
CH = 128
BPW = B // NW
NCHUNK = BPW // CH
NPC = 2 * NCHUNK


def _prep_codes(cur, prev):
    c = jnp.stack([cur, prev]).astype(jnp.int32)
    nb = c.shape[1]
    nchunk = nb // (NW * CH)
    c = c.reshape(2, NW, nchunk, CH, L)
    c = c.transpose(1, 0, 2, 4, 3)
    return c.reshape(NW, 2 * nchunk, L, CH)


@functools.cache
def _build_pool_sc(nb):
    bpw = nb // NW
    nchunk = bpw // CH
    npc = 2 * nchunk
    mesh = plsc.VectorSubcoreMesh(core_axis_name="c", subcore_axis_name="s",
                                  num_cores=NC, num_subcores=NS)

    @functools.partial(
        pl.kernel,
        out_type=jax.ShapeDtypeStruct((4, nb, D), jnp.float32),
        mesh=mesh,
        scratch_types=[
            pltpu.VMEM((2, L, CH), jnp.int32),
            pltpu.VMEM((2, CH, D), jnp.float32),
            pltpu.SemaphoreType.DMA,
            pltpu.SemaphoreType.DMA,
        ],
    )
    def pool_sc(diag_codes_h, proc_codes_h, diag_tab_h, proc_tab_h, out_h,
                idx_v, acc_v, sem0, sem1):
        wid = lax.axis_index("s") * NC + lax.axis_index("c")
        zeros = jnp.zeros((LANES,), jnp.float32)
        sems = (sem0, sem1)

        blocks = [(codes_h, tab_h, t, pc)
                  for t, (codes_h, tab_h) in enumerate(
                      ((diag_codes_h, diag_tab_h), (proc_codes_h, proc_tab_h)))
                  for pc in range(npc)]

        def fire_block(codes_h, tab_h, pc, buf):
            idx = idx_v.at[buf]
            acc = acc_v.at[buf]
            pltpu.sync_copy(codes_h.at[wid, pc], idx)

            def zrow(r, c2):
                for k in range(D // LANES):
                    acc_v[buf, r, pl.ds(k * LANES, LANES)] = zeros
                return c2

            lax.fori_loop(0, CH, zrow, 0)

            def fire(g, c2):
                pltpu.async_copy(tab_h.at[idx.at[g]], acc, sems[buf],
                                 add=True)
                return c2

            lax.fori_loop(0, L, fire, 0)

        def drain_block(tab_h, t, pc, buf):
            idx = idx_v.at[buf]
            acc = acc_v.at[buf]

            def drain(g, c2):
                pltpu.make_async_copy(tab_h.at[idx.at[0]], acc,
                                      sems[buf]).wait()
                return c2

            lax.fori_loop(0, L, drain, 0)
            p = pc // nchunk
            c = pc % nchunk
            base = wid * bpw + c * CH
            pltpu.sync_copy(acc, out_h.at[t * 2 + p, pl.ds(base, CH)])

        for i, (codes_h, tab_h, t, pc) in enumerate(blocks):
            fire_block(codes_h, tab_h, pc, i % 2)
            if i > 0:
                pcodes_h, ptab_h, pt, ppc = blocks[i - 1]
                drain_block(ptab_h, pt, ppc, (i - 1) % 2)
        lcodes_h, ltab_h, lt, lpc = blocks[-1]
        drain_block(ltab_h, lt, lpc, (len(blocks) - 1) % 2)

    return pool_sc


TILE = 2048


def _mlp_body(pooled_ref, w1ta_ref, w1tb_ref, b1_ref, w2ta_ref, w2tb_ref,
              b2_ref, w3t_ref, b3_ref, out_ref):
    ed = pooled_ref[0]
    pd = pooled_ref[1]
    ep = pooled_ref[2]
    pp = pooled_ref[3]
    cur = ed @ w1ta_ref[...] + ep @ w1tb_ref[...] + b1_ref[...]
    prv = pd @ w1ta_ref[...] + pp @ w1tb_ref[...] + b1_ref[...]
    h = jnp.maximum(cur @ w2ta_ref[...] + prv @ w2tb_ref[...] + b2_ref[...],
                    0.0)
    out_ref[...] = jax.nn.sigmoid(h @ w3t_ref[...] + b3_ref[...])


def _mlp_tc(pooled, w1ta, w1tb, b1r, w2ta, w2tb, b2r, w3t, b3r):
    nb = pooled.shape[1]
    return pl.pallas_call(
        _mlp_body,
        grid=(nb // TILE,),
        in_specs=[
            pl.BlockSpec((4, TILE, D), lambda i: (0, i, 0)),
            pl.BlockSpec((D, D), lambda i: (0, 0)),
            pl.BlockSpec((D, D), lambda i: (0, 0)),
            pl.BlockSpec((1, D), lambda i: (0, 0)),
            pl.BlockSpec((D, 2 * D), lambda i: (0, 0)),
            pl.BlockSpec((D, 2 * D), lambda i: (0, 0)),
            pl.BlockSpec((1, 2 * D), lambda i: (0, 0)),
            pl.BlockSpec((2 * D, MED), lambda i: (0, 0)),
            pl.BlockSpec((1, MED), lambda i: (0, 0)),
        ],
        out_specs=pl.BlockSpec((TILE, MED), lambda i: (i, 0)),
        out_shape=jax.ShapeDtypeStruct((nb, MED), jnp.float32),
    )(pooled, w1ta, w1tb, b1r, w2ta, w2tb, b2r, w3t, b3r)


NSPLIT = 1


def kernel(diag_codes, proc_codes, prev_diag_codes, prev_proc_codes,
           diag_table, proc_table, W1, b1, W2, b2, W3, b3):
    h = B // NSPLIT
    w1t, b1r = W1.T, b1.reshape(1, -1)
    w2t, b2r = W2.T, b2.reshape(1, -1)
    w3t, b3r = W3.T, b3.reshape(1, -1)
    w1ta, w1tb = w1t[:D], w1t[D:]
    w2ta, w2tb = w2t[:D], w2t[D:]
    pooled_halves = []
    for s in range(NSPLIT):
        sl = slice(s * h, (s + 1) * h)
        dcodes = _prep_codes(diag_codes[sl], prev_diag_codes[sl])
        pcodes = _prep_codes(proc_codes[sl], prev_proc_codes[sl])
        pooled_halves.append(
            _build_pool_sc(h)(dcodes, pcodes, diag_table, proc_table))
    outs = [_mlp_tc(p, w1ta, w1tb, b1r, w2ta, w2tb, b2r, w3t, b3r)
            for p in pooled_halves]
    return outs[0] if NSPLIT == 1 else jnp.concatenate(outs, axis=0)


# speedup vs baseline: 1.0283x; 1.0054x over previous
import functools

import jax
import jax.numpy as jnp
from jax import lax
from jax.experimental import pallas as pl
from jax.experimental.pallas import tpu as pltpu
from jax.experimental.pallas import tpu_sc as plsc

NC, NS, LANES = 2, 16, 16
NW = NC * NS
B, L, D = 16384, 50, 128
MED = 1000
CH = 128
BPW = B // NW
NCHUNK = BPW // CH
NPC = 2 * NCHUNK


def _prep_codes(cur, prev):
    c = jnp.stack([cur, prev]).astype(jnp.int32)
    nb = c.shape[1]
    nchunk = nb // (NW * CH)
    c = c.reshape(2, NW, nchunk, CH, L)
    c = c.transpose(1, 0, 2, 4, 3)
    return c.reshape(NW, 2 * nchunk, L, CH)


@functools.cache
def _build_pool_sc(nb):
    bpw = nb // NW
    nchunk = bpw // CH
    npc = 2 * nchunk
    mesh = plsc.VectorSubcoreMesh(core_axis_name="c", subcore_axis_name="s",
                                  num_cores=NC, num_subcores=NS)

    @functools.partial(
        pl.kernel,
        out_type=jax.ShapeDtypeStruct((4, nb, D), jnp.float32),
        mesh=mesh,
        scratch_types=[
            pltpu.VMEM((2, L, CH), jnp.int32),
            pltpu.VMEM((2, CH, D), jnp.float32),
            pltpu.SemaphoreType.DMA,
            pltpu.SemaphoreType.DMA,
        ],
    )
    def pool_sc(diag_codes_h, proc_codes_h, diag_tab_h, proc_tab_h, out_h,
                idx_v, acc_v, sem0, sem1):
        wid = lax.axis_index("s") * NC + lax.axis_index("c")
        zeros = jnp.zeros((LANES,), jnp.float32)
        sems = (sem0, sem1)

        blocks = [(codes_h, tab_h, t, pc)
                  for t, (codes_h, tab_h) in enumerate(
                      ((diag_codes_h, diag_tab_h), (proc_codes_h, proc_tab_h)))
                  for pc in range(npc)]

        def fire_block(codes_h, tab_h, pc, buf):
            idx = idx_v.at[buf]
            acc = acc_v.at[buf]
            pltpu.sync_copy(codes_h.at[wid, pc], idx)

            def zrow(r, c2):
                for k in range(D // LANES):
                    acc_v[buf, r, pl.ds(k * LANES, LANES)] = zeros
                return c2

            lax.fori_loop(0, CH, zrow, 0)

            def fire(g, c2):
                pltpu.async_copy(tab_h.at[idx.at[g]], acc, sems[buf],
                                 add=True)
                return c2

            lax.fori_loop(0, L, fire, 0)

        def drain_block(tab_h, t, pc, buf):
            idx = idx_v.at[buf]
            acc = acc_v.at[buf]

            def drain(g, c2):
                pltpu.make_async_copy(tab_h.at[idx.at[0]], acc,
                                      sems[buf]).wait()
                return c2

            lax.fori_loop(0, L, drain, 0)
            p = pc // nchunk
            c = pc % nchunk
            base = wid * bpw + c * CH
            pltpu.sync_copy(acc, out_h.at[t * 2 + p, pl.ds(base, CH)])

        for i, (codes_h, tab_h, t, pc) in enumerate(blocks):
            fire_block(codes_h, tab_h, pc, i % 2)
            if i > 0:
                pcodes_h, ptab_h, pt, ppc = blocks[i - 1]
                drain_block(ptab_h, pt, ppc, (i - 1) % 2)
        lcodes_h, ltab_h, lt, lpc = blocks[-1]
        drain_block(ltab_h, lt, lpc, (len(blocks) - 1) % 2)

    return pool_sc


TILE = 2048


def _dot(a, b):
    return jnp.dot(a, b, preferred_element_type=jnp.float32)


def _mlp_body(pooled_ref, w1ta_ref, w1tb_ref, b1_ref, w2ta_ref, w2tb_ref,
              b2_ref, w3t_ref, b3_ref, out_ref):
    ed = pooled_ref[0].astype(jnp.bfloat16)
    pd = pooled_ref[1].astype(jnp.bfloat16)
    ep = pooled_ref[2].astype(jnp.bfloat16)
    pp = pooled_ref[3].astype(jnp.bfloat16)
    cur = _dot(ed, w1ta_ref[...]) + _dot(ep, w1tb_ref[...]) + b1_ref[...]
    prv = _dot(pd, w1ta_ref[...]) + _dot(pp, w1tb_ref[...]) + b1_ref[...]
    h = jnp.maximum(_dot(cur.astype(jnp.bfloat16), w2ta_ref[...])
                    + _dot(prv.astype(jnp.bfloat16), w2tb_ref[...])
                    + b2_ref[...], 0.0)
    out_ref[...] = jax.nn.sigmoid(
        _dot(h.astype(jnp.bfloat16), w3t_ref[...]) + b3_ref[...])


def _mlp_tc(pooled, w1ta, w1tb, b1r, w2ta, w2tb, b2r, w3t, b3r):
    nb = pooled.shape[1]
    return pl.pallas_call(
        _mlp_body,
        grid=(nb // TILE,),
        in_specs=[
            pl.BlockSpec((4, TILE, D), lambda i: (0, i, 0)),
            pl.BlockSpec((D, D), lambda i: (0, 0)),
            pl.BlockSpec((D, D), lambda i: (0, 0)),
            pl.BlockSpec((1, D), lambda i: (0, 0)),
            pl.BlockSpec((D, 2 * D), lambda i: (0, 0)),
            pl.BlockSpec((D, 2 * D), lambda i: (0, 0)),
            pl.BlockSpec((1, 2 * D), lambda i: (0, 0)),
            pl.BlockSpec((2 * D, MED), lambda i: (0, 0)),
            pl.BlockSpec((1, MED), lambda i: (0, 0)),
        ],
        out_specs=pl.BlockSpec((TILE, MED), lambda i: (i, 0)),
        out_shape=jax.ShapeDtypeStruct((nb, MED), jnp.float32),
    )(pooled, w1ta, w1tb, b1r, w2ta, w2tb, b2r, w3t, b3r)


NSPLIT = 1


def kernel(diag_codes, proc_codes, prev_diag_codes, prev_proc_codes,
           diag_table, proc_table, W1, b1, W2, b2, W3, b3):
    h = B // NSPLIT
    w1t, b1r = W1.T, b1.reshape(1, -1)
    w2t, b2r = W2.T, b2.reshape(1, -1)
    w3t, b3r = W3.T, b3.reshape(1, -1)
    w1ta, w1tb = w1t[:D].astype(jnp.bfloat16), w1t[D:].astype(jnp.bfloat16)
    w2ta, w2tb = w2t[:D].astype(jnp.bfloat16), w2t[D:].astype(jnp.bfloat16)
    w3t = w3t.astype(jnp.bfloat16)
    pooled_halves = []
    for s in range(NSPLIT):
        sl = slice(s * h, (s + 1) * h)
        dcodes = _prep_codes(diag_codes[sl], prev_diag_codes[sl])
        pcodes = _prep_codes(proc_codes[sl], prev_proc_codes[sl])
        pooled_halves.append(
            _build_pool_sc(h)(dcodes, pcodes, diag_table, proc_table))
    outs = [_mlp_tc(p, w1ta, w1tb, b1r, w2ta, w2tb, b2r, w3t, b3r)
            for p in pooled_halves]
    return outs[0] if NSPLIT == 1 else jnp.concatenate(outs, axis=0)
